# baseline (device time: 30287 ns/iter reference)
import jax
import jax.numpy as jnp
from jax import lax
from jax.experimental import pallas as pl
from jax.experimental.pallas import tpu as pltpu

Z = 4


def kernel(partial, resid, gamma):
    _, m, d = partial.shape

    def body(p_ref, r_ref, g_ref, out_ref, comm_ref, send_sems, recv_sems):
        my_x = lax.axis_index("x")
        my_y = lax.axis_index("y")
        my_z = lax.axis_index("z")
        left = (my_z - 1) % Z
        right = (my_z + 1) % Z

        barrier_sem = pltpu.get_barrier_semaphore()
        for nbr in (left, right):
            pl.semaphore_signal(
                barrier_sem, inc=1,
                device_id=(my_x, my_y, nbr),
                device_id_type=pl.DeviceIdType.MESH,
            )
        pl.semaphore_wait(barrier_sem, 2)

        mine = p_ref[0].astype(jnp.bfloat16)
        comm_ref[0] = mine
        acc = mine.astype(jnp.float32)

        for h in range(Z - 1):
            send_slot = h % 2
            recv_slot = (h + 1) % 2
            rdma = pltpu.make_async_remote_copy(
                src_ref=comm_ref.at[send_slot],
                dst_ref=comm_ref.at[recv_slot],
                send_sem=send_sems.at[send_slot],
                recv_sem=recv_sems.at[recv_slot],
                device_id=(my_x, my_y, right),
                device_id_type=pl.DeviceIdType.MESH,
            )
            rdma.start()
            rdma.wait()
            acc = acc + comm_ref[recv_slot].astype(jnp.float32)

        y = acc + r_ref[...]
        rms = jnp.sqrt(jnp.mean(y * y, axis=-1, keepdims=True) + 1e-6)
        out_ref[...] = y / rms * g_ref[...]

    return pl.pallas_call(
        body,
        out_shape=jax.ShapeDtypeStruct((m, d), jnp.float32),
        in_specs=[
            pl.BlockSpec(memory_space=pltpu.VMEM),
            pl.BlockSpec(memory_space=pltpu.VMEM),
            pl.BlockSpec(memory_space=pltpu.VMEM),
        ],
        out_specs=pl.BlockSpec(memory_space=pltpu.VMEM),
        scratch_shapes=[
            pltpu.VMEM((2, m, d), jnp.bfloat16),
            pltpu.SemaphoreType.DMA((2,)),
            pltpu.SemaphoreType.DMA((2,)),
        ],
        compiler_params=pltpu.CompilerParams(collective_id=0),
    )(partial, resid, gamma.reshape(1, d))


# device time: 20137 ns/iter; 1.5040x vs baseline; 1.5040x over previous
import jax
import jax.numpy as jnp
from jax import lax
from jax.experimental import pallas as pl
from jax.experimental.pallas import tpu as pltpu

Z = 4


def kernel(partial, resid, gamma):
    _, m, d = partial.shape
    mq = m // Z

    def body(p_ref, r_ref, g_ref, out_ref,
             my_bf, rs_recv, ag_send, ag_recv,
             rs_send_sems, rs_recv_sems, ag_send_sems, ag_recv_sems):
        my_x = lax.axis_index("x")
        my_y = lax.axis_index("y")
        my_z = lax.axis_index("z")

        barrier_sem = pltpu.get_barrier_semaphore()
        for o in range(1, Z):
            peer = (my_z + o) % Z
            pl.semaphore_signal(
                barrier_sem, inc=1,
                device_id=(my_x, my_y, peer),
                device_id_type=pl.DeviceIdType.MESH,
            )
        pl.semaphore_wait(barrier_sem, Z - 1)

        my_bf[...] = p_ref[...].astype(jnp.bfloat16)

        rs_rdmas = []
        for o in range(1, Z):
            peer = (my_z + o) % Z
            rdma = pltpu.make_async_remote_copy(
                src_ref=my_bf.at[peer],
                dst_ref=rs_recv.at[my_z],
                send_sem=rs_send_sems.at[o],
                recv_sem=rs_recv_sems.at[my_z],
                device_id=(my_x, my_y, peer),
                device_id_type=pl.DeviceIdType.MESH,
            )
            rdma.start()
            rs_rdmas.append(rdma)

        for o in range(1, Z):
            peer = (my_z + o) % Z
            recv = pltpu.make_async_remote_copy(
                src_ref=my_bf.at[peer],
                dst_ref=rs_recv.at[peer],
                send_sem=rs_send_sems.at[o],
                recv_sem=rs_recv_sems.at[peer],
                device_id=(my_x, my_y, peer),
                device_id_type=pl.DeviceIdType.MESH,
            )
            recv.wait_recv()

        acc = my_bf[my_z].astype(jnp.float32)
        for o in range(1, Z):
            acc = acc + rs_recv[(my_z + o) % Z].astype(jnp.float32)

        row0 = my_z * mq
        y = acc + r_ref[pl.ds(row0, mq), :]
        rms = jnp.sqrt(jnp.mean(y * y, axis=-1, keepdims=True) + 1e-6)
        mine = y / rms * g_ref[...]
        out_ref[pl.ds(row0, mq), :] = mine
        ag_send[...] = mine.astype(jnp.bfloat16)

        ag_rdmas = []
        for o in range(1, Z):
            peer = (my_z + o) % Z
            rdma = pltpu.make_async_remote_copy(
                src_ref=ag_send,
                dst_ref=ag_recv.at[my_z],
                send_sem=ag_send_sems.at[o],
                recv_sem=ag_recv_sems.at[my_z],
                device_id=(my_x, my_y, peer),
                device_id_type=pl.DeviceIdType.MESH,
            )
            rdma.start()
            ag_rdmas.append(rdma)

        for o in range(1, Z):
            peer = (my_z + o) % Z
            recv = pltpu.make_async_remote_copy(
                src_ref=ag_send,
                dst_ref=ag_recv.at[peer],
                send_sem=ag_send_sems.at[o],
                recv_sem=ag_recv_sems.at[peer],
                device_id=(my_x, my_y, peer),
                device_id_type=pl.DeviceIdType.MESH,
            )
            recv.wait_recv()
            out_ref[pl.ds(peer * mq, mq), :] = ag_recv[peer].astype(jnp.float32)

        for rdma in rs_rdmas + ag_rdmas:
            rdma.wait_send()

    return pl.pallas_call(
        body,
        out_shape=jax.ShapeDtypeStruct((m, d), jnp.float32),
        in_specs=[
            pl.BlockSpec(memory_space=pltpu.VMEM),
            pl.BlockSpec(memory_space=pltpu.VMEM),
            pl.BlockSpec(memory_space=pltpu.VMEM),
        ],
        out_specs=pl.BlockSpec(memory_space=pltpu.VMEM),
        scratch_shapes=[
            pltpu.VMEM((Z, mq, d), jnp.bfloat16),
            pltpu.VMEM((Z, mq, d), jnp.bfloat16),
            pltpu.VMEM((mq, d), jnp.bfloat16),
            pltpu.VMEM((Z, mq, d), jnp.bfloat16),
            pltpu.SemaphoreType.DMA((Z,)),
            pltpu.SemaphoreType.DMA((Z,)),
            pltpu.SemaphoreType.DMA((Z,)),
            pltpu.SemaphoreType.DMA((Z,)),
        ],
        compiler_params=pltpu.CompilerParams(collective_id=0),
    )(partial.reshape(Z, mq, d), resid, gamma.reshape(1, d))
